# Initial kernel scaffold; baseline (speedup 1.0000x reference)
import jax
import jax.numpy as jnp
from jax.experimental import pallas as pl


def kernel(dynamic_re, static_re, neighbor_matrix):
    raise NotImplementedError("write your pallas kernel here")


# speedup vs baseline: 4340.0636x; 4340.0636x over previous
import jax
import jax.numpy as jnp
from jax import lax
from jax.experimental import pallas as pl
from jax.experimental.pallas import tpu as pltpu

_N = 128
_K = 16


def _trust_body(d_ref, s_ref, nm_ref, nmt_ref, out_ref):
    n = _N
    D = d_ref[...]
    s = s_ref[...]
    ii = lax.broadcasted_iota(jnp.int32, (n, n), 0)
    jj = lax.broadcasted_iota(jnp.int32, (n, n), 1)

    M = jnp.zeros((n, n), jnp.bool_)
    MT = jnp.zeros((n, n), jnp.bool_)
    for k in range(_K):
        col_k = nm_ref[:, k][:, None]
        row_k = nmt_ref[k, :][None, :]
        M = M | (col_k == jj)
        MT = MT | (row_k == ii)
    Mf = M.astype(jnp.float32)
    MTf = MT.astype(jnp.float32)

    NB = jnp.tanh((0.7 * D + 0.3 * s + 0.5) * 0.5)
    CNT = jnp.dot(Mf, MTf, precision=lax.Precision.HIGHEST)
    A = Mf * jnp.where(ii == jj, 1.0, NB)
    F = jnp.broadcast_to(0.5 * s, (n, n))
    diag = ii == jj
    cnt_pos = CNT > 0.0
    inv_cnt = 0.8 / jnp.maximum(CNT, 1.0)

    def wave(m, carry):
        rep, B = carry
        IND = jnp.dot(A, B, precision=lax.Precision.HIGHEST)
        val = jnp.where(M, NB, jnp.where(cnt_pos, IND * inv_cnt, F))
        val = jnp.where(diag, 1.0, val)
        on_l = ((ii == m) & (jj >= m)) | ((jj == m) & (ii >= m))
        rep = jnp.where(on_l, val, rep)
        B = jnp.where(ii == m, rep * MTf, B)
        return rep, B

    rep0 = jnp.zeros((n, n), jnp.float32)
    rep, _ = lax.fori_loop(0, n, wave, (rep0, rep0))
    out_ref[...] = jnp.clip(rep, 0.0, 1.0)


def kernel(dynamic_re, static_re, neighbor_matrix):
    s2 = static_re.reshape(1, _N)
    nmt = neighbor_matrix.T
    return pl.pallas_call(
        _trust_body,
        out_shape=jax.ShapeDtypeStruct((_N, _N), jnp.float32),
    )(dynamic_re, s2, neighbor_matrix, nmt)
